# baseline (speedup 1.0000x reference)
import jax
import jax.numpy as jnp
from jax import lax
from jax.experimental import pallas as pl
from jax.experimental.pallas import tpu as pltpu
from jax.experimental.pallas import tpu_sc as plsc

_NUM_EMBEDDINGS = 1000000
_D = 64
_R = 4
_SCALING = 1.0 / _R
_ROW = 8

_NW = 32
_B = 4096
_L = 50
_BW = _B // _NW
_LC = 5
_NC = _L // _LC
_NBB = _BW // 16


def _adapter_kernel(x_hbm, bt_hbm, a_hbm, out_hbm,
                    xs_v, i8a, i8b,
                    ra0, ra1, ra2, ra3, rb0, rb1, rb2, rb3,
                    bt_v, outh0, outh1, gsa, gsb, osa, osb):
    wid = lax.axis_index("s") * 2 + lax.axis_index("c")
    rows_a = [ra0, ra1, ra2, ra3]
    rows_b = [rb0, rb1, rb2, rb3]

    pltpu.sync_copy(x_hbm.at[pl.ds(wid * (_BW * _L), _BW * _L)], xs_v)
    pltpu.sync_copy(bt_hbm, bt_v)

    i50 = jax.lax.iota(jnp.int32, 16) * _L
    bvecs = [jax.lax.iota(jnp.int32, 16) + bb * 16 for bb in range(_NBB)]
    seven = jnp.full((16,), 7, dtype=jnp.int32)
    rsplat = [jnp.full((16,), r, dtype=jnp.int32) for r in range(_R)]

    def xidx(c, lp, bb):
        l0splat = jnp.full((16,), c * _LC, dtype=jnp.int32)
        return plsc.load_gather(
            xs_v, [i50 + (bb * (16 * _L) + lp) + l0splat])

    def build_lists(c, i8):
        for lp in range(_LC):
            for bb in range(_NBB):
                iv = xidx(c, lp, bb)
                i8[lp, pl.ds(bb * 16, 16)] = lax.shift_right_logical(iv, 3)

    def gather_copies(i8, rows, sem):
        return [pltpu.make_async_copy(a_hbm.at[r].at[i8.at[lp]],
                                      rows[r].at[lp], sem)
                for lp in range(_LC) for r in range(_R)]

    outh = [outh0, outh1]
    osem = [osa, osb]

    def out_copy(c, h, sem):
        return pltpu.make_async_copy(
            outh[h], out_hbm.at[pl.ds(c * _LC, _LC), pl.ds(h * 4, 4),
                                wid, :, :], sem)

    def compute_half(c, rows, h):
        out_v = outh[h]
        for lp in range(_LC):
            lpsplat = jnp.full((16,), lp, dtype=jnp.int32)
            for bh in range(2):
                bbs = range(bh * (_NBB // 2), (bh + 1) * (_NBB // 2))
                lanes = {bb: lax.bitwise_and(xidx(c, lp, bb), seven)
                         for bb in bbs}
                e = [{bb: plsc.load_gather(
                          rows[r], [lpsplat, bvecs[bb], lanes[bb]])
                      for bb in bbs} for r in range(_R)]

                @plsc.parallel_loop(h * 32, h * 32 + 32, unroll=2)
                def d_body(d, _lp=lp, _e=e, _bbs=bbs, _h=h):
                    dsplat = jnp.full((16,), d, dtype=jnp.int32)
                    s = [plsc.load_gather(bt_v, [rsplat[r], dsplat])
                         for r in range(_R)]
                    dt = lax.shift_right_logical(d, 3) - (_h * 4)
                    di = lax.bitwise_and(d, 7)
                    for bb in _bbs:
                        acc = _e[0][bb] * s[0]
                        acc = acc + _e[1][bb] * s[1]
                        acc = acc + _e[2][bb] * s[2]
                        acc = acc + _e[3][bb] * s[3]
                        out_v[_lp, dt, di, pl.ds(bb * 16, 16)] = acc

    build_lists(0, i8a)
    for cp in gather_copies(i8a, rows_a, gsa):
        cp.start()

    def pair_body(i, carry):
        c0 = 2 * i
        c1 = 2 * i + 1
        c2 = lax.min(c1 + 1, _NC - 1)
        build_lists(c1, i8b)
        for cp in gather_copies(i8b, rows_b, gsb):
            cp.start()
        for cp in gather_copies(i8a, rows_a, gsa):
            cp.wait()

        for h in range(2):
            @pl.when(c0 > 0)
            def _(_h=h):
                out_copy(c0 - 1, _h, osem[_h]).wait()
            compute_half(c0, rows_a, h)
            out_copy(c0, h, osem[h]).start()

        build_lists(c2, i8a)
        for cp in gather_copies(i8a, rows_a, gsa):
            cp.start()
        for cp in gather_copies(i8b, rows_b, gsb):
            cp.wait()

        for h in range(2):
            out_copy(c0, h, osem[h]).wait()
            compute_half(c1, rows_b, h)
            out_copy(c1, h, osem[h]).start()
        return carry

    lax.fori_loop(0, _NC // 2, pair_body, 0)
    for cp in gather_copies(i8a, rows_a, gsa):
        cp.wait()
    out_copy(_NC - 1, 0, osa).wait()
    out_copy(_NC - 1, 1, osb).wait()


def kernel(x, A, B):
    bt = (B.T * _SCALING).astype(jnp.float32)
    table = A.reshape(_R, _NUM_EMBEDDINGS // _ROW, _ROW)

    mesh = plsc.VectorSubcoreMesh(core_axis_name="c", subcore_axis_name="s")
    run = pl.kernel(
        _adapter_kernel,
        out_type=jax.ShapeDtypeStruct((_L, _D // 8, _NW, 8, _BW),
                                      jnp.float32),
        mesh=mesh,
        compiler_params=pltpu.CompilerParams(
            needs_layout_passes=False, use_tc_tiling_on_sc=False),
        scratch_types=[
            pltpu.VMEM((_BW * _L,), jnp.int32),
            pltpu.VMEM((_LC, _BW), jnp.int32),
            pltpu.VMEM((_LC, _BW), jnp.int32),
            pltpu.VMEM((_LC, _BW, _ROW), jnp.float32),
            pltpu.VMEM((_LC, _BW, _ROW), jnp.float32),
            pltpu.VMEM((_LC, _BW, _ROW), jnp.float32),
            pltpu.VMEM((_LC, _BW, _ROW), jnp.float32),
            pltpu.VMEM((_LC, _BW, _ROW), jnp.float32),
            pltpu.VMEM((_LC, _BW, _ROW), jnp.float32),
            pltpu.VMEM((_LC, _BW, _ROW), jnp.float32),
            pltpu.VMEM((_LC, _BW, _ROW), jnp.float32),
            pltpu.VMEM((_R, _D), jnp.float32),
            pltpu.VMEM((_LC, 4, 8, _BW), jnp.float32),
            pltpu.VMEM((_LC, 4, 8, _BW), jnp.float32),
            pltpu.SemaphoreType.DMA,
            pltpu.SemaphoreType.DMA,
            pltpu.SemaphoreType.DMA,
            pltpu.SemaphoreType.DMA,
        ],
    )
    out5 = run(x.reshape(_B * _L).astype(jnp.int32), bt, table)
    return out5.transpose(2, 4, 0, 1, 3).reshape(_B, _L, _D)


# speedup vs baseline: 1.0383x; 1.0383x over previous
import jax
import jax.numpy as jnp
from jax import lax
from jax.experimental import pallas as pl
from jax.experimental.pallas import tpu as pltpu
from jax.experimental.pallas import tpu_sc as plsc

_NUM_EMBEDDINGS = 1000000
_D = 64
_R = 4
_SCALING = 1.0 / _R
_ROW = 8

_NW = 32
_B = 4096
_L = 50
_BW = _B // _NW
_LC = 5
_NC = _L // _LC
_NBB = _BW // 16


def _adapter_kernel(x_hbm, bt_hbm, a_hbm, out_hbm,
                    xs_v, i8a, lna, i8b, lnb,
                    ra0, ra1, ra2, ra3, rb0, rb1, rb2, rb3,
                    bt_v, out_v, gsa, gsb, osem):
    wid = lax.axis_index("s") * 2 + lax.axis_index("c")
    rows_a = [ra0, ra1, ra2, ra3]
    rows_b = [rb0, rb1, rb2, rb3]

    pltpu.sync_copy(x_hbm.at[pl.ds(wid * (_BW * _L), _BW * _L)], xs_v)
    pltpu.sync_copy(bt_hbm, bt_v)

    i50 = jax.lax.iota(jnp.int32, 16) * _L
    bvecs = [jax.lax.iota(jnp.int32, 16) + bb * 16 for bb in range(_NBB)]
    seven = jnp.full((16,), 7, dtype=jnp.int32)
    rsplat = [jnp.full((16,), r, dtype=jnp.int32) for r in range(_R)]

    def build_lists(c, i8, ln):
        l0splat = jnp.full((16,), c * _LC, dtype=jnp.int32)
        for lp in range(_LC):
            for bb in range(_NBB):
                pos = i50 + (bb * (16 * _L) + lp)
                iv = plsc.load_gather(xs_v, [pos + l0splat])
                i8[lp, pl.ds(bb * 16, 16)] = lax.shift_right_logical(iv, 3)
                ln[lp, pl.ds(bb * 16, 16)] = lax.bitwise_and(iv, seven)

    def gather_copies(i8, rows, sem):
        return [pltpu.make_async_copy(a_hbm.at[r].at[i8.at[lp]],
                                      rows[r].at[lp], sem)
                for lp in range(_LC) for r in range(_R)]

    def out_copy(c):
        return pltpu.make_async_copy(
            out_v, out_hbm.at[pl.ds(c * _LC, _LC), :, wid, :, :], osem)

    def compute(c, ln, rows):
        for lp in range(_LC):
            lpsplat = jnp.full((16,), lp, dtype=jnp.int32)
            lanes = [ln[lp, pl.ds(bb * 16, 16)] for bb in range(_NBB)]
            e = [[plsc.load_gather(rows[r], [lpsplat, bvecs[bb], lanes[bb]])
                  for bb in range(_NBB)] for r in range(_R)]

            @plsc.parallel_loop(0, _D, unroll=2)
            def d_body(d, _lp=lp, _e=e):
                dsplat = jnp.full((16,), d, dtype=jnp.int32)
                s = [plsc.load_gather(bt_v, [rsplat[r], dsplat])
                     for r in range(_R)]
                dt = lax.shift_right_logical(d, 3)
                di = lax.bitwise_and(d, 7)
                for bb in range(_NBB):
                    acc = _e[0][bb] * s[0]
                    acc = acc + _e[1][bb] * s[1]
                    acc = acc + _e[2][bb] * s[2]
                    acc = acc + _e[3][bb] * s[3]
                    out_v[_lp, dt, di, pl.ds(bb * 16, 16)] = acc
        out_copy(c).start()

    build_lists(0, i8a, lna)
    for cp in gather_copies(i8a, rows_a, gsa):
        cp.start()

    def pair_body(i, carry):
        c0 = 2 * i
        c1 = 2 * i + 1
        c2 = lax.min(c1 + 1, _NC - 1)
        build_lists(c1, i8b, lnb)
        for cp in gather_copies(i8b, rows_b, gsb):
            cp.start()
        for cp in gather_copies(i8a, rows_a, gsa):
            cp.wait()

        @pl.when(i > 0)
        def _():
            out_copy(c0 - 1).wait()
        compute(c0, lna, rows_a)

        build_lists(c2, i8a, lna)
        for cp in gather_copies(i8a, rows_a, gsa):
            cp.start()
        for cp in gather_copies(i8b, rows_b, gsb):
            cp.wait()
        out_copy(c0).wait()
        compute(c1, lnb, rows_b)
        return carry

    lax.fori_loop(0, _NC // 2, pair_body, 0)
    for cp in gather_copies(i8a, rows_a, gsa):
        cp.wait()
    out_copy(_NC - 1).wait()


def kernel(x, A, B):
    xf = x.reshape(_B * _L).astype(jnp.int32)
    bt = (B.T * _SCALING).astype(jnp.float32)
    table = A.reshape(_R, _NUM_EMBEDDINGS // _ROW, _ROW)

    mesh = plsc.VectorSubcoreMesh(core_axis_name="c", subcore_axis_name="s")
    run = pl.kernel(
        _adapter_kernel,
        out_type=jax.ShapeDtypeStruct((_L, _D // 8, _NW, 8, _BW),
                                      jnp.float32),
        mesh=mesh,
        compiler_params=pltpu.CompilerParams(
            needs_layout_passes=False, use_tc_tiling_on_sc=False,
            skip_device_barrier=True),
        scratch_types=[
            pltpu.VMEM((_BW * _L,), jnp.int32),
            pltpu.VMEM((_LC, _BW), jnp.int32),
            pltpu.VMEM((_LC, _BW), jnp.int32),
            pltpu.VMEM((_LC, _BW), jnp.int32),
            pltpu.VMEM((_LC, _BW), jnp.int32),
            pltpu.VMEM((_LC, _BW, _ROW), jnp.float32),
            pltpu.VMEM((_LC, _BW, _ROW), jnp.float32),
            pltpu.VMEM((_LC, _BW, _ROW), jnp.float32),
            pltpu.VMEM((_LC, _BW, _ROW), jnp.float32),
            pltpu.VMEM((_LC, _BW, _ROW), jnp.float32),
            pltpu.VMEM((_LC, _BW, _ROW), jnp.float32),
            pltpu.VMEM((_LC, _BW, _ROW), jnp.float32),
            pltpu.VMEM((_LC, _BW, _ROW), jnp.float32),
            pltpu.VMEM((_R, _D), jnp.float32),
            pltpu.VMEM((_LC, _D // 8, 8, _BW), jnp.float32),
            pltpu.SemaphoreType.DMA,
            pltpu.SemaphoreType.DMA,
            pltpu.SemaphoreType.DMA,
        ],
    )
    out5 = run(xf, bt, table)
    return out5.transpose(2, 4, 0, 1, 3).reshape(_B, _L, _D)
